# baseline (speedup 1.0000x reference)
import functools

import jax
import jax.numpy as jnp
from jax import lax
from jax.experimental import pallas as pl
from jax.experimental.pallas import tpu as pltpu
from jax.experimental.pallas import tpu_sc as plsc

B = 4096
H = 1024
V = 1000
L = 8
EPS = 1e-5

VP = 1024
EMB0 = 8
UROW = EMB0 + VP
M = 5376
BM = 256

NC, NS = 2, 16
NW = NC * NS
BPW = B // NW
NPAIR = BPW // 2
NCHUNK = V // 16


def _tc1_body(x_ref, w_ref, gamma_ref, bias_ref, y_ref, stats_ref):
    i = pl.program_id(0)
    x = x_ref[...]
    rid = i * BM + lax.broadcasted_iota(jnp.int32, (BM, 1), 0)
    scale = jnp.where(rid == 1, 1.0, gamma_ref[...])
    y = lax.dot_general(x * scale, w_ref[...], (((1,), (1,)), ((), ())),
                        preferred_element_type=jnp.float32)
    y = y + jnp.where(rid == 1, bias_ref[...], 0.0)
    y_ref[...] = y
    mean = jnp.mean(x, axis=1, keepdims=True)
    msq = jnp.mean(x * x, axis=1, keepdims=True)
    stats_ref[...] = jnp.concatenate([mean, msq], axis=1)


def _tc2_body(h_ref, e_ref, p_ref):
    p_ref[...] = lax.dot_general(h_ref[...], e_ref[...], (((1,), (1,)), ((), ())),
                                 preferred_element_type=jnp.float32)


def _rsqrt16(x):
    i = plsc.bitcast(x, jnp.int32)
    y = plsc.bitcast(jnp.int32(0x5F3759DF) - (i >> 1), jnp.float32)
    for _ in range(4):
        y = y * (1.5 - 0.5 * x * y * y)
    return y


def _sc_body(y_hbm, p_hbm, stats_hbm, dec_hbm, out_hbm,
             dec_v, stats_v, muq_v, gc_v, a_v, u_v, p_v, o_v, in_sem, out_sem):
    wid = lax.axis_index("s") * NC + lax.axis_index("c")
    b0 = wid * BPW
    pltpu.sync_copy(dec_hbm.at[pl.ds(b0 * L, BPW * L)], dec_v)
    pltpu.sync_copy(stats_hbm.at[pl.ds(0, EMB0 + VP)], stats_v)
    pltpu.sync_copy(stats_hbm.at[pl.ds(UROW + b0, BPW)], muq_v)
    pltpu.sync_copy(y_hbm.at[pl.ds(0, 2)], gc_v)

    lane = lax.iota(jnp.int32, 16)
    hi = lane >> 3
    zero16 = jnp.zeros((16,), jnp.int32)
    one16 = jnp.ones((16,), jnp.int32)

    def pair_body(k, carry):
        s = k & 1
        bg = b0 + 2 * k
        cp_a = pltpu.async_copy(y_hbm.at[dec_v.at[pl.ds(k * 16, 16)]],
                                a_v.at[0], in_sem)
        cp_u = pltpu.async_copy(y_hbm.at[pl.ds(UROW + bg, 2)], u_v.at[0], in_sem)
        cp_p = pltpu.async_copy(p_hbm.at[pl.ds(bg, 2)], p_v.at[0], in_sem)
        cp_a.wait()
        cp_u.wait()
        cp_p.wait()

        t16 = dec_v[pl.ds(k * 16, 16)]
        me = plsc.load_gather(stats_v, [t16, zero16])
        qe = plsc.load_gather(stats_v, [t16, one16])
        bl = 2 * k + hi
        mu = plsc.load_gather(muq_v, [bl, zero16])
        qu = plsc.load_gather(muq_v, [bl, one16])
        pc = plsc.load_gather(p_v, [zero16, hi, t16 - EMB0])
        mean = me + mu
        ex2 = qe + pc * (2.0 / H) + qu
        s1 = _rsqrt16(ex2 - mean * mean + EPS)
        s2 = -mean * s1
        s1l = [s1[l] for l in range(16)]
        s2l = [s2[l] for l in range(16)]

        @pl.when(k >= 2)
        def _():
            pltpu.make_async_copy(out_hbm.at[pl.ds(0, 2 * L * V)],
                                  o_v.at[0], out_sem).wait()

        def chunk(off):
            gv = gc_v[0, pl.ds(off, 16)]
            cv = gc_v[1, pl.ds(off, 16)]
            u0 = u_v[0, 0, pl.ds(off, 16)]
            u1 = u_v[0, 1, pl.ds(off, 16)]
            for l in range(16):
                av = a_v[0, l, pl.ds(off, 16)]
                uv = u0 if l < 8 else u1
                o_v[s, pl.ds(l * V + off, 16)] = (
                    s1l[l] * (av + uv) + s2l[l] * gv + cv)

        def cbody(j, c):
            chunk(j * 16)
            return c

        lax.fori_loop(0, NCHUNK, cbody, 0)
        chunk(V - 16)
        pltpu.async_copy(o_v.at[s], out_hbm.at[pl.ds(bg * L * V, 2 * L * V)],
                         out_sem)
        return carry

    lax.fori_loop(0, NPAIR, pair_body, 0)
    for _ in range(2):
        pltpu.make_async_copy(out_hbm.at[pl.ds(0, 2 * L * V)], o_v.at[0],
                              out_sem).wait()


_tc1 = pl.pallas_call(
    _tc1_body,
    grid=(M // BM,),
    in_specs=[
        pl.BlockSpec((BM, H), lambda i: (i, 0)),
        pl.BlockSpec((VP, H), lambda i: (0, 0)),
        pl.BlockSpec((1, H), lambda i: (0, 0)),
        pl.BlockSpec((1, VP), lambda i: (0, 0)),
    ],
    out_specs=[
        pl.BlockSpec((BM, VP), lambda i: (i, 0)),
        pl.BlockSpec((BM, 2), lambda i: (i, 0)),
    ],
    out_shape=[
        jax.ShapeDtypeStruct((M, VP), jnp.float32),
        jax.ShapeDtypeStruct((M, 2), jnp.float32),
    ],
)

_tc2 = pl.pallas_call(
    _tc2_body,
    grid=(B // BM,),
    in_specs=[
        pl.BlockSpec((BM, H), lambda i: (i, 0)),
        pl.BlockSpec((VP, H), lambda i: (0, 0)),
    ],
    out_specs=pl.BlockSpec((BM, VP), lambda i: (i, 0)),
    out_shape=jax.ShapeDtypeStruct((B, VP), jnp.float32),
)

@functools.lru_cache(maxsize=1)
def _sc_combine():
    return functools.partial(
        pl.kernel,
        out_type=jax.ShapeDtypeStruct((B * L * V,), jnp.float32),
        mesh=plsc.VectorSubcoreMesh(core_axis_name="c", subcore_axis_name="s",
                                    num_cores=NC, num_subcores=NS),
        compiler_params=pltpu.CompilerParams(needs_layout_passes=False,
                                             use_tc_tiling_on_sc=False),
        scratch_types=[
            pltpu.VMEM((BPW * L,), jnp.int32),
            pltpu.VMEM((EMB0 + VP, 2), jnp.float32),
            pltpu.VMEM((BPW, 2), jnp.float32),
            pltpu.VMEM((2, VP), jnp.float32),
            pltpu.VMEM((2, 16, VP), jnp.float32),
            pltpu.VMEM((2, 2, VP), jnp.float32),
            pltpu.VMEM((2, 2, VP), jnp.float32),
            pltpu.VMEM((2, 2 * L * V), jnp.float32),
            pltpu.SemaphoreType.DMA,
            pltpu.SemaphoreType.DMA,
        ],
    )(_sc_body)


def kernel(hidden_states, target_sids, emb_table, ln_gamma, ln_beta, W, b):
    if hidden_states.ndim == 3:
        hidden_states = hidden_states[:, -1, :]
    f32 = jnp.float32
    hidden_states = hidden_states.astype(f32)
    emb_pad = jnp.zeros((VP, H), f32).at[: V + 1].set(emb_table.astype(f32))
    w_pad = jnp.zeros((VP, H), f32).at[:V].set(W.astype(f32))
    b_pad = jnp.zeros((1, VP), f32).at[0, :V].set(b.astype(f32))
    gamma = ln_gamma.astype(f32)[None]
    x = jnp.concatenate([
        jnp.ones((1, H), f32),
        ln_beta.astype(f32)[None],
        jnp.zeros((EMB0 - 2, H), f32),
        emb_pad,
        hidden_states,
        jnp.zeros((M - UROW - B, H), f32),
    ], axis=0)
    dec = jnp.concatenate(
        [jnp.full((B, 1), V, jnp.int32), target_sids[:, :-1].astype(jnp.int32)],
        axis=1) + EMB0
    y, stats = _tc1(x, w_pad, gamma, b_pad)
    p = _tc2(hidden_states, emb_pad)
    out = _sc_combine()(y * 1.0, p * 1.0, stats * 1.0, dec.reshape(-1))
    return out.reshape(B, L, V)


# speedup vs baseline: 1.3064x; 1.3064x over previous
import functools

import jax
import jax.numpy as jnp
from jax import lax
from jax.experimental import pallas as pl
from jax.experimental.pallas import tpu as pltpu
from jax.experimental.pallas import tpu_sc as plsc

B = 4096
H = 1024
V = 1000
L = 8
EPS = 1e-5

VP = 1024
EMB0 = 8
UROW = EMB0 + VP
M = 5376
BM = 256

NC, NS = 2, 16
NW = NC * NS
BPW = B // NW
NPAIR = BPW // 2
NCHUNK = V // 16


def _tc1_body(x_ref, w_ref, gamma_ref, bias_ref, y_ref, stats_ref):
    i = pl.program_id(0)
    x = x_ref[...]
    rid = i * BM + lax.broadcasted_iota(jnp.int32, (BM, 1), 0)
    scale = jnp.where(rid == 1, 1.0, gamma_ref[...])
    y = lax.dot_general(x * scale, w_ref[...], (((1,), (1,)), ((), ())),
                        preferred_element_type=jnp.float32)
    y = y + jnp.where(rid == 1, bias_ref[...], 0.0)
    y_ref[...] = y
    mean = jnp.mean(x, axis=1, keepdims=True)
    msq = jnp.mean(x * x, axis=1, keepdims=True)
    stats_ref[...] = jnp.concatenate([mean, msq], axis=1)


def _tc2_body(h_ref, e_ref, p_ref):
    p_ref[...] = lax.dot_general(h_ref[...], e_ref[...], (((1,), (1,)), ((), ())),
                                 preferred_element_type=jnp.float32)


def _rsqrt16(x):
    i = plsc.bitcast(x, jnp.int32)
    y = plsc.bitcast(jnp.int32(0x5F3759DF) - (i >> 1), jnp.float32)
    for _ in range(4):
        y = y * (1.5 - 0.5 * x * y * y)
    return y


def _sc_body(y_hbm, p_hbm, stats_hbm, dec_hbm, out_hbm,
             dec_v, stats_v, muq_v, gc_v, a_v, u_v, p_v, o_v, in_sem):
    wid = lax.axis_index("s") * NC + lax.axis_index("c")
    b0 = wid * BPW
    pltpu.sync_copy(dec_hbm.at[pl.ds(b0 * L, BPW * L)], dec_v)
    pltpu.sync_copy(stats_hbm.at[pl.ds(0, EMB0 + VP)], stats_v)
    pltpu.sync_copy(stats_hbm.at[pl.ds(UROW + b0, BPW)], muq_v)
    pltpu.sync_copy(y_hbm.at[pl.ds(0, 2)], gc_v)

    lane = lax.iota(jnp.int32, 16)
    hi = lane >> 3
    zero16 = jnp.zeros((16,), jnp.int32)
    one16 = jnp.ones((16,), jnp.int32)

    def pair_body(k, carry):
        bg = b0 + 2 * k
        cp_a = pltpu.async_copy(y_hbm.at[dec_v.at[pl.ds(k * 16, 16)]],
                                a_v, in_sem)
        cp_u = pltpu.async_copy(y_hbm.at[pl.ds(UROW + bg, 2)], u_v, in_sem)
        cp_p = pltpu.async_copy(p_hbm.at[pl.ds(bg, 2)], p_v, in_sem)
        cp_a.wait()
        cp_u.wait()
        cp_p.wait()

        t16 = dec_v[pl.ds(k * 16, 16)]
        me = plsc.load_gather(stats_v, [t16, zero16])
        qe = plsc.load_gather(stats_v, [t16, one16])
        bl = 2 * k + hi
        mu = plsc.load_gather(muq_v, [bl, zero16])
        qu = plsc.load_gather(muq_v, [bl, one16])
        pc = plsc.load_gather(p_v, [hi, t16 - EMB0])
        mean = me + mu
        ex2 = qe + pc * (2.0 / H) + qu
        s1 = _rsqrt16(ex2 - mean * mean + EPS)
        s2 = -mean * s1
        s1l = [s1[l] for l in range(16)]
        s2l = [s2[l] for l in range(16)]

        def chunk(off):
            gv = gc_v[0, pl.ds(off, 16)]
            cv = gc_v[1, pl.ds(off, 16)]
            u0 = u_v[0, pl.ds(off, 16)]
            u1 = u_v[1, pl.ds(off, 16)]
            for l in range(16):
                av = a_v[l, pl.ds(off, 16)]
                uv = u0 if l < 8 else u1
                o_v[pl.ds(l * V + off, 16)] = (
                    s1l[l] * (av + uv) + s2l[l] * gv + cv)

        def cbody(j, c):
            chunk(j * 16)
            return c

        lax.fori_loop(0, NCHUNK, cbody, 0)
        chunk(V - 16)
        pltpu.sync_copy(o_v, out_hbm.at[pl.ds(bg * L * V, 2 * L * V)])
        return carry

    lax.fori_loop(0, NPAIR, pair_body, 0)


_tc1 = pl.pallas_call(
    _tc1_body,
    grid=(M // BM,),
    in_specs=[
        pl.BlockSpec((BM, H), lambda i: (i, 0)),
        pl.BlockSpec((VP, H), lambda i: (0, 0)),
        pl.BlockSpec((1, H), lambda i: (0, 0)),
        pl.BlockSpec((1, VP), lambda i: (0, 0)),
    ],
    out_specs=[
        pl.BlockSpec((BM, VP), lambda i: (i, 0)),
        pl.BlockSpec((BM, 2), lambda i: (i, 0)),
    ],
    out_shape=[
        jax.ShapeDtypeStruct((M, VP), jnp.float32),
        jax.ShapeDtypeStruct((M, 2), jnp.float32),
    ],
)

_tc2 = pl.pallas_call(
    _tc2_body,
    grid=(B // BM,),
    in_specs=[
        pl.BlockSpec((BM, H), lambda i: (i, 0)),
        pl.BlockSpec((VP, H), lambda i: (0, 0)),
    ],
    out_specs=pl.BlockSpec((BM, VP), lambda i: (i, 0)),
    out_shape=jax.ShapeDtypeStruct((B, VP), jnp.float32),
)

@functools.lru_cache(maxsize=1)
def _sc_combine():
    return functools.partial(
        pl.kernel,
        out_type=jax.ShapeDtypeStruct((B * L * V,), jnp.float32),
        mesh=plsc.VectorSubcoreMesh(core_axis_name="c", subcore_axis_name="s",
                                    num_cores=NC, num_subcores=NS),
        compiler_params=pltpu.CompilerParams(needs_layout_passes=False,
                                             use_tc_tiling_on_sc=False),
        scratch_types=[
            pltpu.VMEM((BPW * L,), jnp.int32),
            pltpu.VMEM((EMB0 + VP, 2), jnp.float32),
            pltpu.VMEM((BPW, 2), jnp.float32),
            pltpu.VMEM((2, VP), jnp.float32),
            pltpu.VMEM((16, VP), jnp.float32),
            pltpu.VMEM((2, VP), jnp.float32),
            pltpu.VMEM((2, VP), jnp.float32),
            pltpu.VMEM((2 * L * V,), jnp.float32),
            pltpu.SemaphoreType.DMA,
        ],
    )(_sc_body)


def kernel(hidden_states, target_sids, emb_table, ln_gamma, ln_beta, W, b):
    if hidden_states.ndim == 3:
        hidden_states = hidden_states[:, -1, :]
    f32 = jnp.float32
    hidden_states = hidden_states.astype(f32)
    emb_pad = jnp.zeros((VP, H), f32).at[: V + 1].set(emb_table.astype(f32))
    w_pad = jnp.zeros((VP, H), f32).at[:V].set(W.astype(f32))
    b_pad = jnp.zeros((1, VP), f32).at[0, :V].set(b.astype(f32))
    gamma = ln_gamma.astype(f32)[None]
    x = jnp.concatenate([
        jnp.ones((1, H), f32),
        ln_beta.astype(f32)[None],
        jnp.zeros((EMB0 - 2, H), f32),
        emb_pad,
        hidden_states,
        jnp.zeros((M - UROW - B, H), f32),
    ], axis=0)
    dec = jnp.concatenate(
        [jnp.full((B, 1), V, jnp.int32), target_sids[:, :-1].astype(jnp.int32)],
        axis=1) + EMB0
    y, stats = _tc1(x, w_pad, gamma, b_pad)
    p = _tc2(hidden_states, emb_pad)
    out = _sc_combine()(y * 1.0, p * 1.0, stats * 1.0, dec.reshape(-1))
    return out.reshape(B, L, V)
